# baseline (speedup 1.0000x reference)
import jax
import jax.numpy as jnp
from jax.experimental import pallas as pl

_BLOCK_N = 2048


def _mlp_softmax_kernel(x_ref, w1_ref, b1_ref, w2_ref, b2_ref, out_ref):
    x = x_ref[...]
    h = jnp.dot(x, w1_ref[...], preferred_element_type=jnp.float32) + b1_ref[...]
    h = 0.5 * h * (1.0 + jax.lax.erf(h * 0.7071067811865476))
    logits = jnp.dot(h, w2_ref[...], preferred_element_type=jnp.float32) + b2_ref[...]
    m = jnp.max(logits, axis=1, keepdims=True)
    e = jnp.exp(logits - m)
    out_ref[...] = e / jnp.sum(e, axis=1, keepdims=True)


def kernel(batch_x, W1, b1, W2, b2):
    n, d = batch_x.shape
    h_dim = W1.shape[1]
    q = W2.shape[1]
    grid = (n // _BLOCK_N,)
    return pl.pallas_call(
        _mlp_softmax_kernel,
        grid=grid,
        in_specs=[
            pl.BlockSpec((_BLOCK_N, d), lambda i: (i, 0)),
            pl.BlockSpec((d, h_dim), lambda i: (0, 0)),
            pl.BlockSpec((1, h_dim), lambda i: (0, 0)),
            pl.BlockSpec((h_dim, q), lambda i: (0, 0)),
            pl.BlockSpec((1, q), lambda i: (0, 0)),
        ],
        out_specs=pl.BlockSpec((_BLOCK_N, q), lambda i: (i, 0)),
        out_shape=jax.ShapeDtypeStruct((n, q), jnp.float32),
    )(batch_x, W1, b1.reshape(1, h_dim), W2, b2.reshape(1, q))


# speedup vs baseline: 1.0140x; 1.0140x over previous
import jax
import jax.numpy as jnp
from jax.experimental import pallas as pl

_BLOCK_N = 2048


def _mlp_softmax_kernel(x_ref, w1_ref, b1_ref, w2_ref, b2_ref, out_ref):
    x = x_ref[...]
    ht = jax.lax.dot_general(
        w1_ref[...], x, (((0,), (1,)), ((), ())),
        preferred_element_type=jnp.float32,
    ) + b1_ref[...]
    ht = 0.5 * ht * (1.0 + jax.lax.erf(ht * 0.7071067811865476))
    lt = jax.lax.dot_general(
        w2_ref[...], ht, (((0,), (0,)), ((), ())),
        preferred_element_type=jnp.float32,
    ) + b2_ref[...]
    m = jnp.max(lt, axis=0, keepdims=True)
    e = jnp.exp(lt - m)
    p = e / jnp.sum(e, axis=0, keepdims=True)
    out_ref[...] = p.T


def kernel(batch_x, W1, b1, W2, b2):
    n, d = batch_x.shape
    h_dim = W1.shape[1]
    q = W2.shape[1]
    grid = (n // _BLOCK_N,)
    return pl.pallas_call(
        _mlp_softmax_kernel,
        grid=grid,
        in_specs=[
            pl.BlockSpec((_BLOCK_N, d), lambda i: (i, 0)),
            pl.BlockSpec((d, h_dim), lambda i: (0, 0)),
            pl.BlockSpec((h_dim, 1), lambda i: (0, 0)),
            pl.BlockSpec((h_dim, q), lambda i: (0, 0)),
            pl.BlockSpec((q, 1), lambda i: (0, 0)),
        ],
        out_specs=pl.BlockSpec((_BLOCK_N, q), lambda i: (i, 0)),
        out_shape=jax.ShapeDtypeStruct((n, q), jnp.float32),
    )(batch_x, W1, b1.reshape(h_dim, 1), W2, b2.reshape(q, 1))


# speedup vs baseline: 1.2585x; 1.2412x over previous
import jax
import jax.numpy as jnp
from jax.experimental import pallas as pl

_BLOCK_N = 8192


def _mlp_softmax_kernel(x_ref, w1_ref, b1_ref, w2_ref, b2_ref, out_ref):
    x = x_ref[...]
    ht = jax.lax.dot_general(
        w1_ref[...], x, (((0,), (1,)), ((), ())),
        preferred_element_type=jnp.float32,
    ) + b1_ref[...]
    ht = 0.5 * ht * (1.0 + jax.lax.erf(ht * 0.7071067811865476))
    lt = jax.lax.dot_general(
        w2_ref[...], ht, (((0,), (0,)), ((), ())),
        preferred_element_type=jnp.float32,
    ) + b2_ref[...]
    m = jnp.max(lt, axis=0, keepdims=True)
    e = jnp.exp(lt - m)
    p = e / jnp.sum(e, axis=0, keepdims=True)
    out_ref[...] = p.T


def kernel(batch_x, W1, b1, W2, b2):
    n, d = batch_x.shape
    h_dim = W1.shape[1]
    q = W2.shape[1]
    grid = (n // _BLOCK_N,)
    return pl.pallas_call(
        _mlp_softmax_kernel,
        grid=grid,
        in_specs=[
            pl.BlockSpec((_BLOCK_N, d), lambda i: (i, 0)),
            pl.BlockSpec((d, h_dim), lambda i: (0, 0)),
            pl.BlockSpec((h_dim, 1), lambda i: (0, 0)),
            pl.BlockSpec((h_dim, q), lambda i: (0, 0)),
            pl.BlockSpec((q, 1), lambda i: (0, 0)),
        ],
        out_specs=pl.BlockSpec((_BLOCK_N, q), lambda i: (i, 0)),
        out_shape=jax.ShapeDtypeStruct((n, q), jnp.float32),
    )(batch_x, W1, b1.reshape(h_dim, 1), W2, b2.reshape(q, 1))


# speedup vs baseline: 1.2618x; 1.0026x over previous
import jax
import jax.numpy as jnp
from jax.experimental import pallas as pl
from jax.experimental.pallas import tpu as pltpu

_BLOCK_N = 8192


def _mlp_softmax_kernel(x_ref, w1_ref, b1_ref, w2_ref, b2_ref, out_ref):
    x = x_ref[...]
    ht = jax.lax.dot_general(
        w1_ref[...], x, (((0,), (1,)), ((), ())),
        preferred_element_type=jnp.float32,
    ) + b1_ref[...]
    ht = 0.5 * ht * (1.0 + jax.lax.erf(ht * 0.7071067811865476))
    lt = jax.lax.dot_general(
        w2_ref[...], ht, (((0,), (0,)), ((), ())),
        preferred_element_type=jnp.float32,
    ) + b2_ref[...]
    m = jnp.max(lt, axis=0, keepdims=True)
    e = jnp.exp(lt - m)
    p = e / jnp.sum(e, axis=0, keepdims=True)
    out_ref[...] = p.T


def kernel(batch_x, W1, b1, W2, b2):
    n, d = batch_x.shape
    h_dim = W1.shape[1]
    q = W2.shape[1]
    grid = (n // _BLOCK_N,)
    return pl.pallas_call(
        _mlp_softmax_kernel,
        grid=grid,
        in_specs=[
            pl.BlockSpec((_BLOCK_N, d), lambda i: (i, 0)),
            pl.BlockSpec((d, h_dim), lambda i: (0, 0)),
            pl.BlockSpec((h_dim, 1), lambda i: (0, 0)),
            pl.BlockSpec((h_dim, q), lambda i: (0, 0)),
            pl.BlockSpec((q, 1), lambda i: (0, 0)),
        ],
        out_specs=pl.BlockSpec((_BLOCK_N, q), lambda i: (i, 0)),
        out_shape=jax.ShapeDtypeStruct((n, q), jnp.float32),
        compiler_params=pltpu.CompilerParams(
            dimension_semantics=("parallel",),
        ),
    )(batch_x, W1, b1.reshape(h_dim, 1), W2, b2.reshape(q, 1))
